# baseline (speedup 1.0000x reference)
import functools

import jax
import jax.numpy as jnp
from jax import lax
from jax.experimental import pallas as pl
from jax.experimental.pallas import tpu as pltpu
from jax.experimental.pallas import tpu_sc as plsc

_ROWS = 8192
_D = 768
_BATCH = 4
_NC = 2
_NS = 16
_NW = _NC * _NS
_RPW = _ROWS // _NW
_CH = 32
_NCH = _RPW // _CH
_NBUF = 4

_mesh = plsc.VectorSubcoreMesh(core_axis_name="c", subcore_axis_name="s")


@functools.partial(
    pl.kernel,
    out_type=jax.ShapeDtypeStruct((_BATCH, _ROWS, _D), jnp.float32),
    mesh=_mesh,
    scratch_types=[
        pltpu.VMEM((_NBUF, _CH, _D), jnp.float32),
    ] + [pltpu.SemaphoreType.DMA] * (2 * _NBUF),
)
def _sc_tile_copy(emb_hbm, out_hbm, bufs, *sems):
    rsems = sems[:_NBUF]
    wsems = sems[_NBUF:]
    wid = lax.axis_index("s") * _NC + lax.axis_index("c")
    base = wid * _RPW

    def rd(i):
        return pltpu.make_async_copy(
            emb_hbm.at[pl.ds(base + i * _CH, _CH)], bufs.at[i % _NBUF],
            rsems[i % _NBUF])

    def wr(i, b):
        return pltpu.make_async_copy(
            bufs.at[i % _NBUF], out_hbm.at[b, pl.ds(base + i * _CH, _CH)],
            wsems[i % _NBUF])

    rd(0).start()
    rd(1).start()
    for i in range(_NCH):
        rd(i).wait()
        for b in range(_BATCH):
            wr(i, b).start()
        nxt = i + 2
        if nxt < _NCH:
            prev = nxt - _NBUF
            if prev >= 0:
                for b in range(_BATCH):
                    wr(prev, b).wait()
            rd(nxt).start()
    for i in range(max(0, _NCH - _NBUF), _NCH):
        for b in range(_BATCH):
            wr(i, b).wait()


def kernel(B, T, emb):
    del B, T
    return _sc_tile_copy(emb)


# speedup vs baseline: 1.0350x; 1.0350x over previous
import functools

import jax
import jax.numpy as jnp
from jax import lax
from jax.experimental import pallas as pl
from jax.experimental.pallas import tpu as pltpu
from jax.experimental.pallas import tpu_sc as plsc

_ROWS = 8192
_D = 768
_BATCH = 4
_NC = 2
_NS = 16
_NW = _NC * _NS
_RPW = _ROWS // _NW
_CH = 64
_NCH = _RPW // _CH
_NBUF = 2

_mesh = plsc.VectorSubcoreMesh(core_axis_name="c", subcore_axis_name="s")


@functools.partial(
    pl.kernel,
    out_type=jax.ShapeDtypeStruct((_BATCH, _ROWS, _D), jnp.float32),
    mesh=_mesh,
    scratch_types=[
        pltpu.VMEM((_NBUF, _CH, _D), jnp.float32),
    ] + [pltpu.SemaphoreType.DMA] * (2 * _NBUF),
)
def _sc_tile_copy(emb_hbm, out_hbm, bufs, *sems):
    rsems = sems[:_NBUF]
    wsems = sems[_NBUF:]
    wid = lax.axis_index("s") * _NC + lax.axis_index("c")
    base = wid * _RPW

    def rd(i):
        return pltpu.make_async_copy(
            emb_hbm.at[pl.ds(base + i * _CH, _CH)], bufs.at[i % _NBUF],
            rsems[i % _NBUF])

    def wr(i, b):
        return pltpu.make_async_copy(
            bufs.at[i % _NBUF], out_hbm.at[b, pl.ds(base + i * _CH, _CH)],
            wsems[i % _NBUF])

    rd(0).start()
    rd(1).start()
    for i in range(_NCH):
        rd(i).wait()
        for b in range(_BATCH):
            wr(i, b).start()
        nxt = i + 2
        if nxt < _NCH:
            prev = nxt - _NBUF
            if prev >= 0:
                for b in range(_BATCH):
                    wr(prev, b).wait()
            rd(nxt).start()
    for i in range(max(0, _NCH - _NBUF), _NCH):
        for b in range(_BATCH):
            wr(i, b).wait()


def kernel(B, T, emb):
    del B, T
    return _sc_tile_copy(emb)
